# Initial kernel scaffold; baseline (speedup 1.0000x reference)
import jax
import jax.numpy as jnp
import numpy as np
from jax import lax
from jax.experimental import pallas as pl
from jax.experimental.pallas import tpu as pltpu
from jax.experimental.pallas import tpu_sc as plsc

NUM_BINS = 20
EPS = 1e-8

N_TOTAL = 32 * 512 * 512
NUM_CORES = 2
NUM_SUBCORES = 16
NW = NUM_CORES * NUM_SUBCORES
PER_TILE = N_TOTAL // NW
CHUNK = 16384
NCHUNK = PER_TILE // CHUNK
INNER = CHUNK // 16
ACC_LEN = 768


def _sc_body(p_hbm, t_hbm, out_hbm,
             pbuf0, pbuf1, tbuf0, tbuf1, acc,
             semp0, semp1, semt0, semt1):
    wid = lax.axis_index("s") * NUM_CORES + lax.axis_index("c")
    base = wid * PER_TILE

    lane = lax.iota(jnp.int32, 16)
    ones = jnp.full((16,), 1.0, jnp.float32)
    zeros = jnp.zeros((16,), jnp.float32)

    for k in range(ACC_LEN // 16):
        acc[pl.ds(k * 16, 16)] = zeros

    pbufs = (pbuf0, pbuf1)
    tbufs = (tbuf0, tbuf1)
    semps = (semp0, semp1)
    semts = (semt0, semt1)

    def start(c):
        b = c % 2
        off = pl.multiple_of(base + c * CHUNK, CHUNK)
        hp = pltpu.async_copy(p_hbm.at[pl.ds(off, CHUNK)], pbufs[b], semps[b])
        ht = pltpu.async_copy(t_hbm.at[pl.ds(off, CHUNK)], tbufs[b], semts[b])
        return hp, ht

    def process(pbuf, tbuf):
        def body(i, carry):
            off = pl.multiple_of(i * 16, 16)
            pv = pbuf[pl.ds(off, 16)]
            tv = tbuf[pl.ds(off, 16)]
            idx0 = jnp.clip((pv * 20.0).astype(jnp.int32), 0, NUM_BINS - 1)
            lo_edge = idx0.astype(jnp.float32) * 0.05
            idx = jnp.where(pv < lo_edge, idx0 - 1, idx0)
            idx = jnp.maximum(idx, 0)
            addr = idx * 16 + lane
            plsc.addupdate_scatter(acc, [addr], pv - tv)
            plsc.addupdate_scatter(acc, [addr + 384], ones)
            return carry
        lax.fori_loop(0, INNER, body, 0)

    inflight = {0: start(0), 1: start(1)}
    for c in range(NCHUNK):
        hp, ht = inflight.pop(c)
        hp.wait()
        ht.wait()
        process(pbufs[c % 2], tbufs[c % 2])
        if c + 2 < NCHUNK:
            inflight[c + 2] = start(c + 2)

    pltpu.sync_copy(acc, out_hbm.at[wid])


@jax.jit
def _sc_partials(p, t):
    mesh = plsc.VectorSubcoreMesh(
        core_axis_name="c", subcore_axis_name="s",
        num_cores=NUM_CORES, num_subcores=NUM_SUBCORES)
    fn = pl.kernel(
        _sc_body,
        out_type=jax.ShapeDtypeStruct((NW, ACC_LEN), jnp.float32),
        mesh=mesh,
        scratch_types=[
            pltpu.VMEM((CHUNK,), jnp.float32),
            pltpu.VMEM((CHUNK,), jnp.float32),
            pltpu.VMEM((CHUNK,), jnp.float32),
            pltpu.VMEM((CHUNK,), jnp.float32),
            pltpu.VMEM((ACC_LEN,), jnp.float32),
            pltpu.SemaphoreType.DMA,
            pltpu.SemaphoreType.DMA,
            pltpu.SemaphoreType.DMA,
            pltpu.SemaphoreType.DMA,
        ],
    )
    return fn(p, t)


def _build_fold():
    m = np.zeros((ACC_LEN, 256), np.float32)
    j = np.arange(384)
    b = j // 16
    keep = b < NUM_BINS
    m[j[keep], b[keep]] = 1.0
    m[384 + j[keep], 128 + b[keep]] = 1.0
    return m


_FOLD = _build_fold()


def _tc_final_body(parts_ref, fold_ref, out_ref):
    x = parts_ref[:]
    col = jnp.sum(x, axis=0, keepdims=True)
    r = lax.dot_general(col, fold_ref[:], (((1,), (0,)), ((), ())),
                        preferred_element_type=jnp.float32)
    d = r[:, :128]
    cnt = r[:, 128:]
    safe = jnp.maximum(cnt, 1.0)
    ace = jnp.sum(jnp.where(cnt > 0, jnp.abs(d) / safe, 0.0))
    valid = jnp.sum(jnp.where(cnt > 0, 1.0, 0.0))
    out_ref[0, 0] = ace / (valid + EPS)


@jax.jit
def _tc_final(parts, fold):
    return pl.pallas_call(
        _tc_final_body,
        out_shape=jax.ShapeDtypeStruct((1, 1), jnp.float32),
    )(parts, fold)


def kernel(preds, targets):
    p = preds.reshape(-1)
    t = targets.reshape(-1).astype(jnp.float32)
    parts = _sc_partials(p, t)
    out = _tc_final(parts, jnp.asarray(_FOLD))
    return out[0, 0]


# speedup vs baseline: 1.1546x; 1.1546x over previous
import jax
import jax.numpy as jnp
import numpy as np
from jax import lax
from jax.experimental import pallas as pl
from jax.experimental.pallas import tpu as pltpu
from jax.experimental.pallas import tpu_sc as plsc

NUM_BINS = 20
EPS = 1e-8

N_TOTAL = 32 * 512 * 512
NUM_CORES = 2
NUM_SUBCORES = 16
NW = NUM_CORES * NUM_SUBCORES
PER_TILE = N_TOTAL // NW
CHUNK = 16384
NCHUNK = PER_TILE // CHUNK
INNER = CHUNK // 16
ACC_LEN = 768


def _sc_body(p_hbm, t_hbm, out_hbm,
             pbuf0, pbuf1, tbuf0, tbuf1, acc,
             semp0, semp1, semt0, semt1):
    wid = lax.axis_index("s") * NUM_CORES + lax.axis_index("c")
    base = wid * PER_TILE

    lane = lax.iota(jnp.int32, 16)
    ones = jnp.full((16,), 1.0, jnp.float32)
    zeros = jnp.zeros((16,), jnp.float32)

    for k in range(ACC_LEN // 16):
        acc[pl.ds(k * 16, 16)] = zeros

    pbufs = (pbuf0, pbuf1)
    tbufs = (tbuf0, tbuf1)
    semps = (semp0, semp1)
    semts = (semt0, semt1)

    def start(c):
        b = c % 2
        off = pl.multiple_of(base + c * CHUNK, CHUNK)
        hp = pltpu.async_copy(p_hbm.at[pl.ds(off, CHUNK)], pbufs[b], semps[b])
        ht = pltpu.async_copy(t_hbm.at[pl.ds(off, CHUNK)], tbufs[b], semts[b])
        return hp, ht

    def process(pbuf, tbuf):
        def body(i, carry):
            off = pl.multiple_of(i * 16, 16)
            pv = pbuf[pl.ds(off, 16)]
            tv = tbuf[pl.ds(off, 16)]
            idx0 = jnp.clip((pv * 20.0).astype(jnp.int32), 0, NUM_BINS - 1)
            lo_edge = idx0.astype(jnp.float32) * 0.05
            idx = jnp.where(pv < lo_edge, idx0 - 1, idx0)
            idx = jnp.maximum(idx, 0)
            addr = idx * 16 + lane
            plsc.addupdate_scatter(acc, [addr], pv - tv)
            plsc.addupdate_scatter(acc, [addr + 384], ones)
            return carry
        lax.fori_loop(0, INNER, body, 0)

    inflight = {0: start(0), 1: start(1)}
    for c in range(NCHUNK):
        hp, ht = inflight.pop(c)
        hp.wait()
        ht.wait()
        process(pbufs[c % 2], tbufs[c % 2])
        if c + 2 < NCHUNK:
            inflight[c + 2] = start(c + 2)

    pltpu.sync_copy(acc, out_hbm.at[wid])


@jax.jit
def _sc_partials(p, t):
    mesh = plsc.VectorSubcoreMesh(
        core_axis_name="c", subcore_axis_name="s",
        num_cores=NUM_CORES, num_subcores=NUM_SUBCORES)
    fn = pl.kernel(
        _sc_body,
        out_type=jax.ShapeDtypeStruct((NW, ACC_LEN), jnp.float32),
        mesh=mesh,
        compiler_params=pltpu.CompilerParams(needs_layout_passes=False),
        scratch_types=[
            pltpu.VMEM((CHUNK,), jnp.float32),
            pltpu.VMEM((CHUNK,), jnp.float32),
            pltpu.VMEM((CHUNK,), jnp.float32),
            pltpu.VMEM((CHUNK,), jnp.float32),
            pltpu.VMEM((ACC_LEN,), jnp.float32),
            pltpu.SemaphoreType.DMA,
            pltpu.SemaphoreType.DMA,
            pltpu.SemaphoreType.DMA,
            pltpu.SemaphoreType.DMA,
        ],
    )
    return fn(p, t)


def _build_fold():
    m = np.zeros((ACC_LEN, 256), np.float32)
    j = np.arange(384)
    b = j // 16
    keep = b < NUM_BINS
    m[j[keep], b[keep]] = 1.0
    m[384 + j[keep], 128 + b[keep]] = 1.0
    return m


_FOLD = _build_fold()


def _tc_final_body(parts_ref, fold_ref, out_ref):
    x = parts_ref[:]
    col = jnp.sum(x, axis=0, keepdims=True)
    r = lax.dot_general(col, fold_ref[:], (((1,), (0,)), ((), ())),
                        preferred_element_type=jnp.float32)
    d = r[:, :128]
    cnt = r[:, 128:]
    safe = jnp.maximum(cnt, 1.0)
    ace = jnp.sum(jnp.where(cnt > 0, jnp.abs(d) / safe, 0.0))
    valid = jnp.sum(jnp.where(cnt > 0, 1.0, 0.0))
    out_ref[:] = (ace / (valid + EPS)).reshape(1, 1)


@jax.jit
def _tc_final(parts, fold):
    return pl.pallas_call(
        _tc_final_body,
        out_shape=jax.ShapeDtypeStruct((1, 1), jnp.float32),
    )(parts, fold)


def kernel(preds, targets):
    p = preds.reshape(-1)
    t = targets.reshape(-1).astype(jnp.float32)
    parts = _sc_partials(p, t)
    out = _tc_final(parts, jnp.asarray(_FOLD))
    return out[0, 0]


# speedup vs baseline: 3.1612x; 2.7380x over previous
import jax
import jax.numpy as jnp
import numpy as np
from jax import lax
from jax.experimental import pallas as pl
from jax.experimental.pallas import tpu as pltpu
from jax.experimental.pallas import tpu_sc as plsc

NUM_BINS = 20
EPS = 1e-8

N_TOTAL = 32 * 512 * 512
NUM_CORES = 2
NUM_SUBCORES = 16
NW = NUM_CORES * NUM_SUBCORES
PER_TILE = N_TOTAL // NW
CHUNK = 16384
NCHUNK = PER_TILE // CHUNK
INNER = CHUNK // 16
ACC_HALF = 384
ACC_LEN = 768


def _sc_body(p_hbm, t_hbm, out_hbm,
             pbuf0, pbuf1, tbuf0, tbuf1, acc_d, acc_c,
             semp0, semp1, semt0, semt1):
    wid = lax.axis_index("s") * NUM_CORES + lax.axis_index("c")
    base = wid * PER_TILE

    lane = lax.iota(jnp.int32, 16)
    ones = jnp.full((16,), 1.0, jnp.float32)
    zeros = jnp.zeros((16,), jnp.float32)

    for k in range(ACC_HALF // 16):
        acc_d[pl.ds(k * 16, 16)] = zeros
        acc_c[pl.ds(k * 16, 16)] = zeros

    pbufs = (pbuf0, pbuf1)
    tbufs = (tbuf0, tbuf1)
    semps = (semp0, semp1)
    semts = (semt0, semt1)

    def start(c):
        b = c % 2
        off = pl.multiple_of(base + c * CHUNK, CHUNK)
        hp = pltpu.async_copy(p_hbm.at[pl.ds(off, CHUNK)], pbufs[b], semps[b])
        ht = pltpu.async_copy(t_hbm.at[pl.ds(off, CHUNK)], tbufs[b], semts[b])
        return hp, ht

    def process(pbuf, tbuf):
        @plsc.parallel_loop(0, CHUNK, step=16, unroll=8)
        def body(off):
            pv = pbuf[pl.ds(off, 16)]
            tv = tbuf[pl.ds(off, 16)]
            idx0 = jnp.minimum((pv * 20.0).astype(jnp.int32), NUM_BINS - 1)
            lo_edge = idx0.astype(jnp.float32) * 0.05
            idx = jnp.where(pv < lo_edge, idx0 - 1, idx0)
            idx = jnp.maximum(idx, 0)
            addr = idx * 16 + lane
            plsc.addupdate_scatter(acc_d, [addr], pv - tv)
            plsc.addupdate_scatter(acc_c, [addr], ones)

    inflight = {0: start(0), 1: start(1)}
    for c in range(NCHUNK):
        hp, ht = inflight.pop(c)
        hp.wait()
        ht.wait()
        process(pbufs[c % 2], tbufs[c % 2])
        if c + 2 < NCHUNK:
            inflight[c + 2] = start(c + 2)

    pltpu.sync_copy(acc_d, out_hbm.at[wid, 0])
    pltpu.sync_copy(acc_c, out_hbm.at[wid, 1])


@jax.jit
def _sc_partials(p, t):
    mesh = plsc.VectorSubcoreMesh(
        core_axis_name="c", subcore_axis_name="s",
        num_cores=NUM_CORES, num_subcores=NUM_SUBCORES)
    fn = pl.kernel(
        _sc_body,
        out_type=jax.ShapeDtypeStruct((NW, 2, ACC_HALF), jnp.float32),
        mesh=mesh,
        compiler_params=pltpu.CompilerParams(needs_layout_passes=False),
        scratch_types=[
            pltpu.VMEM((CHUNK,), jnp.float32),
            pltpu.VMEM((CHUNK,), jnp.float32),
            pltpu.VMEM((CHUNK,), jnp.float32),
            pltpu.VMEM((CHUNK,), jnp.float32),
            pltpu.VMEM((ACC_HALF,), jnp.float32),
            pltpu.VMEM((ACC_HALF,), jnp.float32),
            pltpu.SemaphoreType.DMA,
            pltpu.SemaphoreType.DMA,
            pltpu.SemaphoreType.DMA,
            pltpu.SemaphoreType.DMA,
        ],
    )
    return fn(p, t)


def _build_fold():
    m = np.zeros((ACC_LEN, 256), np.float32)
    j = np.arange(384)
    b = j // 16
    keep = b < NUM_BINS
    m[j[keep], b[keep]] = 1.0
    m[384 + j[keep], 128 + b[keep]] = 1.0
    return m


_FOLD = _build_fold()


def _tc_final_body(parts_ref, fold_ref, out_ref):
    x = parts_ref[:]
    col = jnp.sum(x, axis=0, keepdims=True)
    r = lax.dot_general(col, fold_ref[:], (((1,), (0,)), ((), ())),
                        preferred_element_type=jnp.float32)
    d = r[:, :128]
    cnt = r[:, 128:]
    safe = jnp.maximum(cnt, 1.0)
    ace = jnp.sum(jnp.where(cnt > 0, jnp.abs(d) / safe, 0.0))
    valid = jnp.sum(jnp.where(cnt > 0, 1.0, 0.0))
    out_ref[:] = (ace / (valid + EPS)).reshape(1, 1)


@jax.jit
def _tc_final(parts, fold):
    return pl.pallas_call(
        _tc_final_body,
        out_shape=jax.ShapeDtypeStruct((1, 1), jnp.float32),
    )(parts, fold)


def kernel(preds, targets):
    p = preds.reshape(-1)
    t = targets.reshape(-1).astype(jnp.float32)
    parts = _sc_partials(p, t).reshape(NW, ACC_LEN)
    out = _tc_final(parts, jnp.asarray(_FOLD))
    return out[0, 0]


# speedup vs baseline: 3.2762x; 1.0364x over previous
import jax
import jax.numpy as jnp
import numpy as np
from jax import lax
from jax.experimental import pallas as pl
from jax.experimental.pallas import tpu as pltpu
from jax.experimental.pallas import tpu_sc as plsc

NUM_BINS = 20
EPS = 1e-8

N_TOTAL = 32 * 512 * 512
NUM_CORES = 2
NUM_SUBCORES = 16
NW = NUM_CORES * NUM_SUBCORES
PER_TILE = N_TOTAL // NW
CHUNK = 16384
NCHUNK = PER_TILE // CHUNK
INNER = CHUNK // 16
ACC_HALF = 384
ACC_LEN = 768


def _sc_body(p_hbm, t_hbm, out_hbm,
             pbuf0, pbuf1, tbuf0, tbuf1, acc_d, acc_c,
             semp0, semp1, semt0, semt1):
    wid = lax.axis_index("s") * NUM_CORES + lax.axis_index("c")
    base = wid * PER_TILE

    lane = lax.iota(jnp.int32, 16)
    ones = jnp.full((16,), 1.0, jnp.float32)
    zeros = jnp.zeros((16,), jnp.float32)

    for k in range(ACC_HALF // 16):
        acc_d[pl.ds(k * 16, 16)] = zeros
        acc_c[pl.ds(k * 16, 16)] = zeros

    pbufs = (pbuf0, pbuf1)
    tbufs = (tbuf0, tbuf1)
    semps = (semp0, semp1)
    semts = (semt0, semt1)

    def start(c):
        b = c % 2
        off = pl.multiple_of(base + c * CHUNK, CHUNK)
        hp = pltpu.async_copy(p_hbm.at[pl.ds(off, CHUNK)], pbufs[b], semps[b])
        ht = pltpu.async_copy(t_hbm.at[pl.ds(off, CHUNK)], tbufs[b], semts[b])
        return hp, ht

    def process(pbuf, tbuf):
        @plsc.parallel_loop(0, CHUNK, step=16, unroll=16)
        def body(off):
            pv = pbuf[pl.ds(off, 16)]
            tv = tbuf[pl.ds(off, 16)]
            fm = jnp.minimum(pv * 20.0, 19.0)
            idx0 = fm.astype(jnp.int32)
            lo_edge = idx0.astype(jnp.float32) * 0.05
            idx = jnp.where(pv < lo_edge, idx0 - 1, idx0)
            addr = idx * 16 + lane
            plsc.addupdate_scatter(acc_d, [addr], pv - tv)
            plsc.addupdate_scatter(acc_c, [addr], ones)

    inflight = {0: start(0), 1: start(1)}
    for c in range(NCHUNK):
        hp, ht = inflight.pop(c)
        hp.wait()
        ht.wait()
        process(pbufs[c % 2], tbufs[c % 2])
        if c + 2 < NCHUNK:
            inflight[c + 2] = start(c + 2)

    pltpu.sync_copy(acc_d, out_hbm.at[wid, 0])
    pltpu.sync_copy(acc_c, out_hbm.at[wid, 1])


@jax.jit
def _sc_partials(p, t):
    mesh = plsc.VectorSubcoreMesh(
        core_axis_name="c", subcore_axis_name="s",
        num_cores=NUM_CORES, num_subcores=NUM_SUBCORES)
    fn = pl.kernel(
        _sc_body,
        out_type=jax.ShapeDtypeStruct((NW, 2, ACC_HALF), jnp.float32),
        mesh=mesh,
        compiler_params=pltpu.CompilerParams(needs_layout_passes=False),
        scratch_types=[
            pltpu.VMEM((CHUNK,), jnp.float32),
            pltpu.VMEM((CHUNK,), jnp.float32),
            pltpu.VMEM((CHUNK,), jnp.float32),
            pltpu.VMEM((CHUNK,), jnp.float32),
            pltpu.VMEM((ACC_HALF,), jnp.float32),
            pltpu.VMEM((ACC_HALF,), jnp.float32),
            pltpu.SemaphoreType.DMA,
            pltpu.SemaphoreType.DMA,
            pltpu.SemaphoreType.DMA,
            pltpu.SemaphoreType.DMA,
        ],
    )
    return fn(p, t)


def _build_fold():
    m = np.zeros((ACC_LEN, 256), np.float32)
    j = np.arange(384)
    b = j // 16
    keep = b < NUM_BINS
    m[j[keep], b[keep]] = 1.0
    m[384 + j[keep], 128 + b[keep]] = 1.0
    return m


_FOLD = _build_fold()


def _tc_final_body(parts_ref, fold_ref, out_ref):
    x = parts_ref[:]
    col = jnp.sum(x, axis=0, keepdims=True)
    r = lax.dot_general(col, fold_ref[:], (((1,), (0,)), ((), ())),
                        preferred_element_type=jnp.float32)
    d = r[:, :128]
    cnt = r[:, 128:]
    safe = jnp.maximum(cnt, 1.0)
    ace = jnp.sum(jnp.where(cnt > 0, jnp.abs(d) / safe, 0.0))
    valid = jnp.sum(jnp.where(cnt > 0, 1.0, 0.0))
    out_ref[:] = (ace / (valid + EPS)).reshape(1, 1)


@jax.jit
def _tc_final(parts, fold):
    return pl.pallas_call(
        _tc_final_body,
        out_shape=jax.ShapeDtypeStruct((1, 1), jnp.float32),
    )(parts, fold)


def kernel(preds, targets):
    p = preds.reshape(-1)
    t = targets.reshape(-1).astype(jnp.float32)
    parts = _sc_partials(p, t).reshape(NW, ACC_LEN)
    out = _tc_final(parts, jnp.asarray(_FOLD))
    return out[0, 0]


# speedup vs baseline: 5.7214x; 1.7463x over previous
import jax
import jax.numpy as jnp
import numpy as np
from jax import lax
from jax.experimental import pallas as pl
from jax.experimental.pallas import tpu as pltpu
from jax.experimental.pallas import tpu_sc as plsc

NUM_BINS = 20
EPS = 1e-8

N_TOTAL = 32 * 512 * 512
NUM_CORES = 2
NUM_SUBCORES = 16
NW = NUM_CORES * NUM_SUBCORES
PER_TILE = N_TOTAL // NW
CHUNK = 16384
ROWS_PER_CHUNK = CHUNK // 512
NCHUNK = PER_TILE // CHUNK
INNER = CHUNK // 16
ACC_HALF = 384
ACC_LEN = 768


def _sc_body(p_hbm, t_hbm, out_hbm,
             pbuf0, pbuf1, tbuf0, tbuf1, acc_d, acc_c,
             semp0, semp1, semt0, semt1):
    wid = lax.axis_index("s") * NUM_CORES + lax.axis_index("c")

    lane = lax.iota(jnp.int32, 16)
    ones = jnp.full((16,), 1.0, jnp.float32)
    zeros = jnp.zeros((16,), jnp.float32)

    for k in range(ACC_HALF // 16):
        acc_d[pl.ds(k * 16, 16)] = zeros
        acc_c[pl.ds(k * 16, 16)] = zeros

    pbufs = (pbuf0, pbuf1)
    tbufs = (tbuf0, tbuf1)
    semps = (semp0, semp1)
    semts = (semt0, semt1)

    def start(c):
        b = c % 2
        rows = pl.multiple_of(c * ROWS_PER_CHUNK, ROWS_PER_CHUNK)
        hp = pltpu.async_copy(
            p_hbm.at[wid, 0, pl.ds(rows, ROWS_PER_CHUNK), :], pbufs[b], semps[b])
        ht = pltpu.async_copy(
            t_hbm.at[wid, 0, pl.ds(rows, ROWS_PER_CHUNK), :], tbufs[b], semts[b])
        return hp, ht

    def process(pbuf, tbuf):
        @plsc.parallel_loop(0, CHUNK, step=16, unroll=16)
        def body(off):
            r = off >> 9
            col = off & 511
            pv = pbuf[r, pl.ds(col, 16)]
            tv = tbuf[r, pl.ds(col, 16)]
            fm = jnp.minimum(pv * 20.0, 19.0)
            idx0 = fm.astype(jnp.int32)
            lo_edge = idx0.astype(jnp.float32) * 0.05
            idx = jnp.where(pv < lo_edge, idx0 - 1, idx0)
            addr = idx * 16 + lane
            plsc.addupdate_scatter(acc_d, [addr], pv - tv)
            plsc.addupdate_scatter(acc_c, [addr], ones)

    inflight = {0: start(0), 1: start(1)}
    for c in range(NCHUNK):
        hp, ht = inflight.pop(c)
        hp.wait()
        ht.wait()
        process(pbufs[c % 2], tbufs[c % 2])
        if c + 2 < NCHUNK:
            inflight[c + 2] = start(c + 2)

    pltpu.sync_copy(acc_d, out_hbm.at[wid, 0])
    pltpu.sync_copy(acc_c, out_hbm.at[wid, 1])


@jax.jit
def _sc_partials(p, t):
    mesh = plsc.VectorSubcoreMesh(
        core_axis_name="c", subcore_axis_name="s",
        num_cores=NUM_CORES, num_subcores=NUM_SUBCORES)
    fn = pl.kernel(
        _sc_body,
        out_type=jax.ShapeDtypeStruct((NW, 2, ACC_HALF), jnp.float32),
        mesh=mesh,
        compiler_params=pltpu.CompilerParams(needs_layout_passes=False),
        scratch_types=[
            pltpu.VMEM((ROWS_PER_CHUNK, 512), jnp.float32),
            pltpu.VMEM((ROWS_PER_CHUNK, 512), jnp.float32),
            pltpu.VMEM((ROWS_PER_CHUNK, 512), jnp.float32),
            pltpu.VMEM((ROWS_PER_CHUNK, 512), jnp.float32),
            pltpu.VMEM((ACC_HALF,), jnp.float32),
            pltpu.VMEM((ACC_HALF,), jnp.float32),
            pltpu.SemaphoreType.DMA,
            pltpu.SemaphoreType.DMA,
            pltpu.SemaphoreType.DMA,
            pltpu.SemaphoreType.DMA,
        ],
    )
    return fn(p, t)


def _build_fold():
    m = np.zeros((ACC_LEN, 256), np.float32)
    j = np.arange(384)
    b = j // 16
    keep = b < NUM_BINS
    m[j[keep], b[keep]] = 1.0
    m[384 + j[keep], 128 + b[keep]] = 1.0
    return m


_FOLD = _build_fold()


def _tc_final_body(parts_ref, fold_ref, out_ref):
    x = parts_ref[:]
    col = jnp.sum(x, axis=0, keepdims=True)
    r = lax.dot_general(col, fold_ref[:], (((1,), (0,)), ((), ())),
                        preferred_element_type=jnp.float32)
    d = r[:, :128]
    cnt = r[:, 128:]
    safe = jnp.maximum(cnt, 1.0)
    ace = jnp.sum(jnp.where(cnt > 0, jnp.abs(d) / safe, 0.0))
    valid = jnp.sum(jnp.where(cnt > 0, 1.0, 0.0))
    out_ref[:] = (ace / (valid + EPS)).reshape(1, 1)


@jax.jit
def _tc_final(parts, fold):
    return pl.pallas_call(
        _tc_final_body,
        out_shape=jax.ShapeDtypeStruct((1, 1), jnp.float32),
    )(parts, fold)


def kernel(preds, targets):
    parts = _sc_partials(preds, targets.astype(jnp.float32))
    parts = parts.reshape(NW, ACC_LEN)
    out = _tc_final(parts, jnp.asarray(_FOLD))
    return out[0, 0]
